# baseline (speedup 1.0000x reference)
import jax
import jax.numpy as jnp
from jax import lax
from jax.experimental import pallas as pl
from jax.experimental.pallas import tpu as pltpu
from jax.experimental.pallas import tpu_sc as plsc

N = 10000
NP = 10240
D = 128
E = 320000

NC = 2
NS = 16
NW = NC * NS
CH = 128
CPT = 80
EP = NW * CPT * CH
RPT = NP // NS

_f32 = jnp.float32
_i32 = jnp.int32


def _spmm_body(h_hbm, src_hbm, dst_hbm, w_hbm, out_hbm,
               src_v, dst_v, w_v, rows_v, sem, acc):
    c = lax.axis_index("c")
    s = lax.axis_index("s")
    wid = s * NC + c

    pltpu.sync_copy(src_hbm.at[wid], src_v)
    pltpu.sync_copy(dst_hbm.at[wid], dst_v)
    pltpu.sync_copy(w_hbm.at[wid], w_v)

    zero16 = jnp.zeros((16,), _f32)

    def _zero_buf(i, carry):
        r = i // 8
        k = i % 8
        rows_v[r, pl.ds(k * 16, 16)] = zero16
        return carry

    lax.fori_loop(0, CH * 8, _zero_buf, 0)
    row0 = s * RPT
    for k in range(RPT // CH):
        pltpu.sync_copy(rows_v, acc.at[pl.ds(row0 + k * CH, CH)])
    plsc.subcore_barrier()

    def _chunk(j, carry):
        pltpu.async_copy(h_hbm.at[src_v.at[j]], rows_v, sem).wait()

        def _edge16(b, carry2):
            wv16 = w_v[j, pl.ds(b * 16, 16)]
            for e2 in range(16):
                e = b * 16 + e2
                wspl = lax.gather(
                    wv16, jnp.full((16, 1), e2, _i32),
                    dimension_numbers=lax.GatherDimensionNumbers(
                        offset_dims=(), collapsed_slice_dims=(0,),
                        start_index_map=(0,)),
                    slice_sizes=(1,),
                    mode=lax.GatherScatterMode.PROMISE_IN_BOUNDS)
                for k in range(D // 16):
                    sl = pl.ds(k * 16, 16)
                    rows_v[e, sl] = rows_v[e, sl] * wspl
            return carry2

        lax.fori_loop(0, CH // 16, _edge16, 0)
        pltpu.sync_copy(rows_v, acc.at[dst_v.at[j]], add=True)
        return carry

    lax.fori_loop(0, CPT, _chunk, 0)
    plsc.subcore_barrier()

    pltpu.sync_copy(acc.at[pl.ds(row0, RPT)],
                    out_hbm.at[c, pl.ds(row0, RPT)])


_spmm = pl.kernel(
    _spmm_body,
    out_type=jax.ShapeDtypeStruct((NC, NP, D), _f32),
    mesh=plsc.VectorSubcoreMesh(core_axis_name="c", subcore_axis_name="s"),
    scratch_types=[
        pltpu.VMEM((CPT, CH), _i32),
        pltpu.VMEM((CPT, CH), _i32),
        pltpu.VMEM((CPT, CH), _f32),
        pltpu.VMEM((CH, D), _f32),
        pltpu.SemaphoreType.DMA,
        pltpu.VMEM_SHARED((NP, D), _f32),
    ],
)


_RB = 1024


def _mm_body(x_ref, w_ref, o_ref):
    o_ref[...] = jnp.dot(x_ref[...], w_ref[...],
                         preferred_element_type=_f32)


def _mm2_body(p_ref, w_ref, o_ref):
    h = jnp.tanh(p_ref[0] + p_ref[1])
    o_ref[...] = jnp.dot(h, w_ref[...], preferred_element_type=_f32)


def _norm_body(p_ref, o_ref):
    t = jnp.tanh(p_ref[0] + p_ref[1])
    sq = jnp.sum(t * t, axis=1, keepdims=True)
    o_ref[...] = t * lax.rsqrt(jnp.maximum(sq, 1e-12))


_mm = pl.pallas_call(
    _mm_body,
    grid=(NP // _RB,),
    in_specs=[pl.BlockSpec((_RB, D), lambda i: (i, 0)),
              pl.BlockSpec((D, D), lambda i: (0, 0))],
    out_specs=pl.BlockSpec((_RB, D), lambda i: (i, 0)),
    out_shape=jax.ShapeDtypeStruct((NP, D), _f32),
)

_mm2 = pl.pallas_call(
    _mm2_body,
    grid=(NP // _RB,),
    in_specs=[pl.BlockSpec((NC, _RB, D), lambda i: (0, i, 0)),
              pl.BlockSpec((D, D), lambda i: (0, 0))],
    out_specs=pl.BlockSpec((_RB, D), lambda i: (i, 0)),
    out_shape=jax.ShapeDtypeStruct((NP, D), _f32),
)

_norm = pl.pallas_call(
    _norm_body,
    grid=(NP // _RB,),
    in_specs=[pl.BlockSpec((NC, _RB, D), lambda i: (0, i, 0))],
    out_specs=pl.BlockSpec((_RB, D), lambda i: (i, 0)),
    out_shape=jax.ShapeDtypeStruct((NP, D), _f32),
)


def kernel(input_embed, edge_index, edge_weight, W0, W1):
    pad = EP - E
    src = jnp.concatenate([edge_index[0], jnp.zeros((pad,), _i32)])
    dst = jnp.concatenate([edge_index[1], jnp.zeros((pad,), _i32)])
    w = jnp.concatenate([edge_weight, jnp.zeros((pad,), _f32)])
    src = src.reshape(NW, CPT, CH)
    dst = dst.reshape(NW, CPT, CH)
    w = w.reshape(NW, CPT, CH)

    x = jnp.concatenate(
        [input_embed, jnp.zeros((NP - N, D), _f32)], axis=0)

    h0 = _mm(x, W0)
    p0 = _spmm(h0, src, dst, w)
    h1 = _mm2(p0, W1)
    p1 = _spmm(h1, src, dst, w)
    return _norm(p1)[:N]


# speedup vs baseline: 2.3500x; 2.3500x over previous
import jax
import jax.numpy as jnp
from jax import lax
from jax.experimental import pallas as pl
from jax.experimental.pallas import tpu as pltpu
from jax.experimental.pallas import tpu_sc as plsc

N = 10000
NP = 10240
D = 128
E = 320000

NC = 2
NS = 16
NW = NC * NS
CH = 128
CPT = 80
EP = NW * CPT * CH
RPT = NP // NS

_f32 = jnp.float32
_i32 = jnp.int32


def _spmm_body(h_hbm, src_hbm, dst_hbm, w_hbm, out_hbm,
               src_v, dst_v, w_v, rows_v, sem, acc):
    c = lax.axis_index("c")
    s = lax.axis_index("s")
    wid = s * NC + c

    pltpu.sync_copy(src_hbm.at[wid], src_v)
    pltpu.sync_copy(dst_hbm.at[wid], dst_v)
    pltpu.sync_copy(w_hbm.at[wid], w_v)

    zero16 = jnp.zeros((16,), _f32)

    def _zero_buf(i, carry):
        r = i // 8
        k = i % 8
        rows_v[r, pl.ds(k * 16, 16)] = zero16
        return carry

    lax.fori_loop(0, CH * 8, _zero_buf, 0)
    row0 = s * RPT
    for k in range(RPT // CH):
        pltpu.sync_copy(rows_v, acc.at[pl.ds(row0 + k * CH, CH)])
    plsc.subcore_barrier()

    def _chunk(j, carry):
        pltpu.async_copy(h_hbm.at[src_v.at[j]], rows_v, sem).wait()

        def _edge16(b, carry2):
            wv16 = w_v[j, pl.ds(b * 16, 16)]
            for e2 in range(16):
                e = b * 16 + e2
                wspl = lax.gather(
                    wv16, jnp.full((16, 1), e2, _i32),
                    dimension_numbers=lax.GatherDimensionNumbers(
                        offset_dims=(), collapsed_slice_dims=(0,),
                        start_index_map=(0,)),
                    slice_sizes=(1,),
                    mode=lax.GatherScatterMode.PROMISE_IN_BOUNDS)
                for k in range(D // 16):
                    sl = pl.ds(k * 16, 16)
                    rows_v[e, sl] = rows_v[e, sl] * wspl
            return carry2

        lax.fori_loop(0, CH // 16, _edge16, 0)
        pltpu.sync_copy(rows_v, acc.at[dst_v.at[j]], add=True)
        return carry

    lax.fori_loop(0, CPT, _chunk, 0)
    plsc.subcore_barrier()

    pltpu.sync_copy(acc.at[pl.ds(row0, RPT)],
                    out_hbm.at[c, pl.ds(row0, RPT)])


_spmm = pl.kernel(
    _spmm_body,
    out_type=jax.ShapeDtypeStruct((NC, NP, D), _f32),
    mesh=plsc.VectorSubcoreMesh(core_axis_name="c", subcore_axis_name="s"),
    scratch_types=[
        pltpu.VMEM((CPT, CH), _i32),
        pltpu.VMEM((CPT, CH), _i32),
        pltpu.VMEM((CPT, CH), _f32),
        pltpu.VMEM((CH, D), _f32),
        pltpu.SemaphoreType.DMA,
        pltpu.VMEM_SHARED((NP, D), _f32),
    ],
)


_RB = 1024


def _mm_body(x_ref, w_ref, o_ref):
    o_ref[...] = jnp.dot(x_ref[...], w_ref[...],
                         preferred_element_type=_f32)


def _mm2_body(p_ref, w_ref, o_ref):
    h = jnp.tanh(p_ref[0] + p_ref[1])
    o_ref[...] = jnp.dot(h, w_ref[...], preferred_element_type=_f32)


def _norm_body(p_ref, o_ref):
    t = jnp.tanh(p_ref[0] + p_ref[1])
    sq = jnp.sum(t * t, axis=1, keepdims=True)
    o_ref[...] = t * lax.rsqrt(jnp.maximum(sq, 1e-12))


_mm = pl.pallas_call(
    _mm_body,
    grid=(NP // _RB,),
    in_specs=[pl.BlockSpec((_RB, D), lambda i: (i, 0)),
              pl.BlockSpec((D, D), lambda i: (0, 0))],
    out_specs=pl.BlockSpec((_RB, D), lambda i: (i, 0)),
    out_shape=jax.ShapeDtypeStruct((NP, D), _f32),
)

_mm2 = pl.pallas_call(
    _mm2_body,
    grid=(NP // _RB,),
    in_specs=[pl.BlockSpec((NC, _RB, D), lambda i: (0, i, 0)),
              pl.BlockSpec((D, D), lambda i: (0, 0))],
    out_specs=pl.BlockSpec((_RB, D), lambda i: (i, 0)),
    out_shape=jax.ShapeDtypeStruct((NP, D), _f32),
)

_norm = pl.pallas_call(
    _norm_body,
    grid=(NP // _RB,),
    in_specs=[pl.BlockSpec((NC, _RB, D), lambda i: (0, i, 0))],
    out_specs=pl.BlockSpec((_RB, D), lambda i: (i, 0)),
    out_shape=jax.ShapeDtypeStruct((NP, D), _f32),
)


def kernel(input_embed, edge_index, edge_weight, W0, W1):
    pad = EP - E
    spread = (jnp.arange(pad, dtype=_i32) * 8) % NP
    src = jnp.concatenate([edge_index[0], spread])
    dst = jnp.concatenate([edge_index[1], spread])
    w = jnp.concatenate([edge_weight, jnp.zeros((pad,), _f32)])
    src = src.reshape(NW, CPT, CH)
    dst = dst.reshape(NW, CPT, CH)
    w = w.reshape(NW, CPT, CH)

    x = jnp.concatenate(
        [input_embed, jnp.zeros((NP - N, D), _f32)], axis=0)

    h0 = _mm(x, W0)
    p0 = _spmm(h0, src, dst, w)
    h1 = _mm2(p0, W1)
    p1 = _spmm(h1, src, dst, w)
    return _norm(p1)[:N]


# speedup vs baseline: 3.5559x; 1.5132x over previous
import jax
import jax.numpy as jnp
from jax import lax
from jax.experimental import pallas as pl
from jax.experimental.pallas import tpu as pltpu
from jax.experimental.pallas import tpu_sc as plsc

N = 10000
NP = 10240
D = 128
E = 320000

NC = 2
NS = 16
NW = NC * NS
CH = 128
CPT = 80
EP = NW * CPT * CH
RPT = NP // NS

_f32 = jnp.float32
_i32 = jnp.int32


HCPT = CPT // 2
NPH = 2


def _spmm_body(h_hbm, src_hbm, dst_hbm, w_hbm, out_hbm,
               src_v, dst_v, w_v, rows0, rows1, gs0, gs1, ss0, ss1, acc):
    c = lax.axis_index("c")
    s = lax.axis_index("s")
    wid = s * NC + c

    zero16 = jnp.zeros((16,), _f32)

    def _zero_buf(i, carry):
        r = i // 8
        k = i % 8
        rows0[r, pl.ds(k * 16, 16)] = zero16
        return carry

    lax.fori_loop(0, CH * 8, _zero_buf, 0)
    row0 = s * RPT
    for k in range(RPT // CH):
        pltpu.sync_copy(rows0, acc.at[pl.ds(row0 + k * CH, CH)])
    plsc.subcore_barrier()

    def _scale(rows_v, j):
        def _edge16(b, carry2):
            wv16 = w_v[j, pl.ds(b * 16, 16)]
            for e2 in range(16):
                e = b * 16 + e2
                wspl = lax.gather(
                    wv16, jnp.full((16, 1), e2, _i32),
                    dimension_numbers=lax.GatherDimensionNumbers(
                        offset_dims=(), collapsed_slice_dims=(0,),
                        start_index_map=(0,)),
                    slice_sizes=(1,),
                    mode=lax.GatherScatterMode.PROMISE_IN_BOUNDS)
                for k in range(D // 16):
                    sl = pl.ds(k * 16, 16)
                    rows_v[e, sl] = rows_v[e, sl] * wspl
            return carry2

        lax.fori_loop(0, CH // 16, _edge16, 0)

    for half in range(NPH):
        pltpu.sync_copy(src_hbm.at[wid, pl.ds(half * HCPT, HCPT)], src_v)
        pltpu.sync_copy(dst_hbm.at[wid, pl.ds(half * HCPT, HCPT)], dst_v)
        pltpu.sync_copy(w_hbm.at[wid, pl.ds(half * HCPT, HCPT)], w_v)
        pltpu.async_copy(h_hbm.at[src_v.at[0]], rows0, gs0)

        def _step(j2, carry):
            j = 2 * j2

            @pl.when(j2 >= 1)
            def _():
                pltpu.make_async_copy(
                    rows1, acc.at[dst_v.at[j - 1]], ss1).wait()

            pltpu.async_copy(h_hbm.at[src_v.at[j + 1]], rows1, gs1)
            pltpu.make_async_copy(
                h_hbm.at[src_v.at[j]], rows0, gs0).wait()
            _scale(rows0, j)
            pltpu.async_copy(rows0, acc.at[dst_v.at[j]], ss0, add=True)

            pltpu.make_async_copy(
                rows0, acc.at[dst_v.at[j]], ss0).wait()

            @pl.when(j2 < HCPT // 2 - 1)
            def _():
                pltpu.async_copy(h_hbm.at[src_v.at[j + 2]], rows0, gs0)

            pltpu.make_async_copy(
                h_hbm.at[src_v.at[j + 1]], rows1, gs1).wait()
            _scale(rows1, j + 1)
            pltpu.async_copy(rows1, acc.at[dst_v.at[j + 1]], ss1, add=True)
            return carry

        lax.fori_loop(0, HCPT // 2, _step, 0)
        pltpu.make_async_copy(
            rows1, acc.at[dst_v.at[HCPT - 1]], ss1).wait()

    plsc.subcore_barrier()

    pltpu.sync_copy(acc.at[pl.ds(row0, RPT)],
                    out_hbm.at[c, pl.ds(row0, RPT)])


_spmm = pl.kernel(
    _spmm_body,
    out_type=jax.ShapeDtypeStruct((NC, NP, D), _f32),
    mesh=plsc.VectorSubcoreMesh(core_axis_name="c", subcore_axis_name="s"),
    scratch_types=[
        pltpu.VMEM((HCPT, CH), _i32),
        pltpu.VMEM((HCPT, CH), _i32),
        pltpu.VMEM((HCPT, CH), _f32),
        pltpu.VMEM((CH, D), _f32),
        pltpu.VMEM((CH, D), _f32),
        pltpu.SemaphoreType.DMA,
        pltpu.SemaphoreType.DMA,
        pltpu.SemaphoreType.DMA,
        pltpu.SemaphoreType.DMA,
        pltpu.VMEM_SHARED((NP, D), _f32),
    ],
)


_RB = 1024


def _mm_body(x_ref, w_ref, o_ref):
    o_ref[...] = jnp.dot(x_ref[...], w_ref[...],
                         preferred_element_type=_f32)


def _mm2_body(p_ref, w_ref, o_ref):
    h = jnp.tanh(p_ref[0] + p_ref[1])
    o_ref[...] = jnp.dot(h, w_ref[...], preferred_element_type=_f32)


def _norm_body(p_ref, o_ref):
    t = jnp.tanh(p_ref[0] + p_ref[1])
    sq = jnp.sum(t * t, axis=1, keepdims=True)
    o_ref[...] = t * lax.rsqrt(jnp.maximum(sq, 1e-12))


_mm = pl.pallas_call(
    _mm_body,
    grid=(NP // _RB,),
    in_specs=[pl.BlockSpec((_RB, D), lambda i: (i, 0)),
              pl.BlockSpec((D, D), lambda i: (0, 0))],
    out_specs=pl.BlockSpec((_RB, D), lambda i: (i, 0)),
    out_shape=jax.ShapeDtypeStruct((NP, D), _f32),
)

_mm2 = pl.pallas_call(
    _mm2_body,
    grid=(NP // _RB,),
    in_specs=[pl.BlockSpec((NC, _RB, D), lambda i: (0, i, 0)),
              pl.BlockSpec((D, D), lambda i: (0, 0))],
    out_specs=pl.BlockSpec((_RB, D), lambda i: (i, 0)),
    out_shape=jax.ShapeDtypeStruct((NP, D), _f32),
)

_norm = pl.pallas_call(
    _norm_body,
    grid=(NP // _RB,),
    in_specs=[pl.BlockSpec((NC, _RB, D), lambda i: (0, i, 0))],
    out_specs=pl.BlockSpec((_RB, D), lambda i: (i, 0)),
    out_shape=jax.ShapeDtypeStruct((NP, D), _f32),
)


def kernel(input_embed, edge_index, edge_weight, W0, W1):
    pad = EP - E
    spread = (jnp.arange(pad, dtype=_i32) * 8) % NP
    src = jnp.concatenate([edge_index[0], spread])
    dst = jnp.concatenate([edge_index[1], spread])
    w = jnp.concatenate([edge_weight, jnp.zeros((pad,), _f32)])
    src = src.reshape(NW, CPT, CH)
    dst = dst.reshape(NW, CPT, CH)
    w = w.reshape(NW, CPT, CH)

    x = jnp.concatenate(
        [input_embed, jnp.zeros((NP - N, D), _f32)], axis=0)

    h0 = _mm(x, W0)
    p0 = _spmm(h0, src, dst, w)
    h1 = _mm2(p0, W1)
    p1 = _spmm(h1, src, dst, w)
    return _norm(p1)[:N]
